# baseline (speedup 1.0000x reference)
import functools

import jax
import jax.numpy as jnp
from jax import lax
from jax.experimental import pallas as pl
from jax.experimental.pallas import tpu as pltpu
from jax.experimental.pallas import tpu_sc as plsc

E = 160000
D = 128
S = 5
R = 2 * S + 2
C = 40
G = 4
GROWS = R * C // G
NCH = E // C
NW = 32
CPW = NCH // NW
VPR = D // 16


def _make_sc_kernel():
    mesh = plsc.VectorSubcoreMesh(core_axis_name="c", subcore_axis_name="s")

    @functools.partial(
        pl.kernel,
        mesh=mesh,
        out_type=(
            jax.ShapeDtypeStruct((NCH, 2 * C, D), jnp.float32),
            jax.ShapeDtypeStruct((NCH, 2 * C, D), jnp.float32),
        ),
        scratch_types=[
            pltpu.VMEM((G, GROWS), jnp.int32),
            pltpu.VMEM((G * GROWS, D), jnp.float32),
            pltpu.VMEM((2 * C, D), jnp.float32),
            pltpu.SemaphoreType.DMA,
            pltpu.SemaphoreType.DMA,
        ],
    )
    def k(x_hbm, idx_hbm, ea_hbm, nb_hbm, idx_v, buf, nbo, gsem, osem):
        wid = lax.axis_index("s") * 2 + lax.axis_index("c")

        def chunk_body(j, carry):
            ch = wid * CPW + j
            pltpu.sync_copy(idx_hbm.at[ch], idx_v)
            gathers = [
                pltpu.async_copy(
                    x_hbm.at[idx_v.at[g]],
                    buf.at[pl.ds(g * GROWS, GROWS)],
                    gsem,
                )
                for g in range(G)
            ]
            for cp in gathers:
                cp.wait()
            out_ea = pltpu.async_copy(buf.at[pl.ds(0, 2 * C)],
                                      ea_hbm.at[ch], osem)

            def cbody(c, cc):
                for v in range(VPR):
                    sl = pl.ds(16 * v, 16)
                    s_ = buf[2 * c, sl]
                    d_ = buf[2 * c + 1, sl]
                    nbo[2 * c, sl] = (s_ + d_) * 0.5
                    acc = buf[2 * C + c, sl]
                    for r in range(1, 2 * S):
                        acc = acc + buf[2 * C + r * C + c, sl]
                    nbo[2 * c + 1, sl] = acc * jnp.float32(1.0 / (2 * S))
                return cc

            lax.fori_loop(0, C, cbody, 0)

            pltpu.sync_copy(nbo, nb_hbm.at[ch])
            out_ea.wait()
            return carry

        lax.fori_loop(0, CPW, chunk_body, 0)

    return k


_sc_agg = _make_sc_kernel()


def kernel(x, edge_index, nb_idx):
    src = edge_index[0]
    dst = edge_index[1]
    ed_inter = jnp.stack([src, dst], axis=-1).reshape(NCH, 2 * C)
    nb_part = (
        jnp.concatenate([jnp.transpose(nb_idx[0]),
                         jnp.transpose(nb_idx[1])], axis=0)
        .reshape(2 * S, NCH, C)
        .transpose(1, 0, 2)
        .reshape(NCH, 2 * S * C)
    )
    idx_ch = jnp.concatenate([ed_inter, nb_part], axis=1).reshape(
        NCH, G, GROWS)
    ea, nb = _sc_agg(x, idx_ch)
    return ea.reshape(E, 2 * D), nb.reshape(E, 2 * D)


# speedup vs baseline: 1.0011x; 1.0011x over previous
import functools

import jax
import jax.numpy as jnp
from jax import lax
from jax.experimental import pallas as pl
from jax.experimental.pallas import tpu as pltpu
from jax.experimental.pallas import tpu_sc as plsc

E = 160000
D = 128
S = 5
R = 2 * S + 2
C = 40
G = 4
GROWS = R * C // G
NCH = E // C
NW = 32
CPW = NCH // NW
VPR = D // 16


def _make_sc_kernel():
    mesh = plsc.VectorSubcoreMesh(core_axis_name="c", subcore_axis_name="s")

    @functools.partial(
        pl.kernel,
        mesh=mesh,
        out_type=(
            jax.ShapeDtypeStruct((2 * E, D), jnp.float32),
            jax.ShapeDtypeStruct((2 * E, D), jnp.float32),
        ),
        scratch_types=[
            pltpu.VMEM((G, GROWS), jnp.int32),
            pltpu.VMEM((G * GROWS, D), jnp.float32),
            pltpu.VMEM((2 * C, D), jnp.float32),
            pltpu.SemaphoreType.DMA,
            pltpu.SemaphoreType.DMA,
        ],
    )
    def k(x_hbm, idx_hbm, ea_hbm, nb_hbm, idx_v, buf, nbo, gsem, osem):
        wid = lax.axis_index("s") * 2 + lax.axis_index("c")

        def chunk_body(j, carry):
            ch = wid * CPW + j
            pltpu.sync_copy(idx_hbm.at[ch], idx_v)
            gathers = [
                pltpu.async_copy(
                    x_hbm.at[idx_v.at[g]],
                    buf.at[pl.ds(g * GROWS, GROWS)],
                    gsem,
                )
                for g in range(G)
            ]
            for cp in gathers:
                cp.wait()
            out_ea = pltpu.async_copy(buf.at[pl.ds(0, 2 * C)],
                                      ea_hbm.at[pl.ds(ch * 2 * C, 2 * C)],
                                      osem)

            def cbody(c, cc):
                for v in range(VPR):
                    sl = pl.ds(16 * v, 16)
                    s_ = buf[2 * c, sl]
                    d_ = buf[2 * c + 1, sl]
                    nbo[2 * c, sl] = (s_ + d_) * 0.5
                    acc = buf[2 * C + c, sl]
                    for r in range(1, 2 * S):
                        acc = acc + buf[2 * C + r * C + c, sl]
                    nbo[2 * c + 1, sl] = acc * jnp.float32(1.0 / (2 * S))
                return cc

            lax.fori_loop(0, C, cbody, 0)

            pltpu.sync_copy(nbo, nb_hbm.at[pl.ds(ch * 2 * C, 2 * C)])
            out_ea.wait()
            return carry

        lax.fori_loop(0, CPW, chunk_body, 0)

    return k


_sc_agg = _make_sc_kernel()


def kernel(x, edge_index, nb_idx):
    src = edge_index[0]
    dst = edge_index[1]
    ed_inter = jnp.stack([src, dst], axis=-1).reshape(NCH, 2 * C)
    nb_part = (
        jnp.concatenate([jnp.transpose(nb_idx[0]),
                         jnp.transpose(nb_idx[1])], axis=0)
        .reshape(2 * S, NCH, C)
        .transpose(1, 0, 2)
        .reshape(NCH, 2 * S * C)
    )
    idx_ch = jnp.concatenate([ed_inter, nb_part], axis=1).reshape(
        NCH, G, GROWS)
    ea, nb = _sc_agg(x, idx_ch)
    return ea.reshape(E, 2 * D), nb.reshape(E, 2 * D)


# speedup vs baseline: 1.8234x; 1.8214x over previous
import functools

import jax
import jax.numpy as jnp
from jax import lax
from jax.experimental import pallas as pl
from jax.experimental.pallas import tpu as pltpu
from jax.experimental.pallas import tpu_sc as plsc

E = 160000
D = 128
S = 5
R = 2 * S + 2
C = 40
G = 4
GROWS = R * C // G
NCH = E // C
NW = 32
CPW = NCH // NW


def _make_sc_kernel():
    mesh = plsc.VectorSubcoreMesh(core_axis_name="c", subcore_axis_name="s")

    @functools.partial(
        pl.kernel,
        mesh=mesh,
        out_type=(
            jax.ShapeDtypeStruct((E, 2 * D), jnp.float32),
            jax.ShapeDtypeStruct((E, 2 * D), jnp.float32),
        ),
        scratch_types=[
            pltpu.VMEM((G, GROWS), jnp.int32),
            pltpu.VMEM((R * C, D), jnp.float32),
            pltpu.VMEM((C, D), jnp.float32),
            pltpu.VMEM((C, D), jnp.float32),
            pltpu.SemaphoreType.DMA,
            pltpu.SemaphoreType.DMA,
        ],
    )
    def k(x_hbm, idx_hbm, ea_hbm, nb_hbm, idx_v, buf, nbl, nbr, gsem, osem):
        wid = lax.axis_index("s") * 2 + lax.axis_index("c")

        def chunk_body(j, carry):
            chunk = wid * CPW + j
            base = chunk * C
            pltpu.sync_copy(idx_hbm.at[chunk], idx_v)
            gathers = [
                pltpu.async_copy(
                    x_hbm.at[idx_v.at[g]],
                    buf.at[pl.ds(g * GROWS, GROWS)],
                    gsem,
                )
                for g in range(G)
            ]
            for cp in gathers:
                cp.wait()
            out1 = pltpu.async_copy(
                buf.at[pl.ds(0, C)],
                ea_hbm.at[pl.ds(base, C), pl.ds(0, D)],
                osem,
            )
            out2 = pltpu.async_copy(
                buf.at[pl.ds(C, C)],
                ea_hbm.at[pl.ds(base, C), pl.ds(D, D)],
                osem,
            )

            def cbody(c, cc):
                for v in range(D // 16):
                    sl = pl.ds(v * 16, 16)
                    s_ = buf[c, sl]
                    d_ = buf[C + c, sl]
                    nbl[c, sl] = (s_ + d_) * 0.5
                    acc = buf[2 * C + c, sl]
                    for r in range(3, R):
                        acc = acc + buf[r * C + c, sl]
                    nbr[c, sl] = acc * jnp.float32(1.0 / (2 * S))
                return cc

            lax.fori_loop(0, C, cbody, 0)

            pltpu.sync_copy(nbl, nb_hbm.at[pl.ds(base, C), pl.ds(0, D)])
            pltpu.sync_copy(nbr, nb_hbm.at[pl.ds(base, C), pl.ds(D, D)])
            out1.wait()
            out2.wait()
            return carry

        lax.fori_loop(0, CPW, chunk_body, 0)

    return k


_sc_agg = _make_sc_kernel()


def kernel(x, edge_index, nb_idx):
    src = edge_index[0]
    dst = edge_index[1]
    idx_full = jnp.concatenate(
        [src[None, :], dst[None, :],
         jnp.transpose(nb_idx[0]), jnp.transpose(nb_idx[1])],
        axis=0,
    )
    idx_ch = (
        idx_full.reshape(R, NCH, C)
        .transpose(1, 0, 2)
        .reshape(NCH, G, GROWS)
    )
    ea, nb = _sc_agg(x, idx_ch)
    return ea, nb


# speedup vs baseline: 1.8447x; 1.0117x over previous
import functools

import jax
import jax.numpy as jnp
from jax import lax
from jax.experimental import pallas as pl
from jax.experimental.pallas import tpu as pltpu
from jax.experimental.pallas import tpu_sc as plsc

E = 160000
D = 128
S = 5
R = 2 * S + 2
C = 40
G = 4
GROWS = R * C // G
NCH = E // C
NW = 32
CPW = NCH // NW


def _make_sc_kernel():
    mesh = plsc.VectorSubcoreMesh(core_axis_name="c", subcore_axis_name="s")

    @functools.partial(
        pl.kernel,
        mesh=mesh,
        out_type=(
            jax.ShapeDtypeStruct((E, 2 * D), jnp.float32),
            jax.ShapeDtypeStruct((E, 2 * D), jnp.float32),
        ),
        scratch_types=[
            pltpu.VMEM((G, GROWS), jnp.int32),
            pltpu.VMEM((R * C, D), jnp.float32),
            pltpu.VMEM((C, D), jnp.float32),
            pltpu.VMEM((C, D), jnp.float32),
            pltpu.SemaphoreType.DMA,
            pltpu.SemaphoreType.DMA,
        ],
    )
    def k(x_hbm, idx_hbm, ea_hbm, nb_hbm, idx_v, buf, nbl, nbr, gsem, osem):
        wid = lax.axis_index("s") * 2 + lax.axis_index("c")

        def chunk_body(j, carry):
            chunk = wid * CPW + j
            base = chunk * C
            pltpu.sync_copy(idx_hbm.at[chunk], idx_v)
            gathers = [
                pltpu.async_copy(
                    x_hbm.at[idx_v.at[g]],
                    buf.at[pl.ds(g * GROWS, GROWS)],
                    gsem,
                )
                for g in range(G)
            ]
            for cp in gathers:
                cp.wait()
            out1 = pltpu.async_copy(
                buf.at[pl.ds(0, C)],
                ea_hbm.at[pl.ds(base, C), pl.ds(0, D)],
                osem,
            )
            out2 = pltpu.async_copy(
                buf.at[pl.ds(C, C)],
                ea_hbm.at[pl.ds(base, C), pl.ds(D, D)],
                osem,
            )

            def cbody(c, cc):
                for v in range(D // 16):
                    sl = pl.ds(v * 16, 16)
                    s_ = buf[c, sl]
                    d_ = buf[C + c, sl]
                    nbl[c, sl] = (s_ + d_) * 0.5
                    acc = buf[2 * C + c, sl]
                    for r in range(3, R):
                        acc = acc + buf[r * C + c, sl]
                    nbr[c, sl] = acc * jnp.float32(1.0 / (2 * S))
                return cc

            lax.fori_loop(0, C, cbody, 0)

            out3 = pltpu.async_copy(
                nbl, nb_hbm.at[pl.ds(base, C), pl.ds(0, D)], osem)
            out4 = pltpu.async_copy(
                nbr, nb_hbm.at[pl.ds(base, C), pl.ds(D, D)], osem)
            out1.wait()
            out2.wait()
            out3.wait()
            out4.wait()
            return carry

        lax.fori_loop(0, CPW, chunk_body, 0)

    return k


_sc_agg = _make_sc_kernel()


def kernel(x, edge_index, nb_idx):
    src = edge_index[0]
    dst = edge_index[1]
    idx_full = jnp.concatenate(
        [src[None, :], dst[None, :],
         jnp.transpose(nb_idx[0]), jnp.transpose(nb_idx[1])],
        axis=0,
    )
    idx_ch = (
        idx_full.reshape(R, NCH, C)
        .transpose(1, 0, 2)
        .reshape(NCH, G, GROWS)
    )
    ea, nb = _sc_agg(x, idx_ch)
    return ea, nb
